# baseline (speedup 1.0000x reference)
import functools

import jax
import jax.numpy as jnp
from jax import lax
from jax.experimental import pallas as pl
from jax.experimental.pallas import tpu as pltpu
from jax.experimental.pallas import tpu_sc as plsc

N = 10000
E = 320000
D = 128
NUM_GRAPHS = 64
PPAD = 16

_NC, _NS = 2, 16
_NW = _NC * _NS
_W = 128
_NWIN = E // _W
_NKW = 2496 // _NW
_PAIRS = _NKW // 2
_NLEFT = _NWIN - _NKW * _NW
_WE = 64
_NWINE = E // _WE
_NKWE = (_NWINE // _NW) & ~3
_QUADS = _NKWE // 4
_NLEFTE = _NWINE - _NKWE * _NW
_RPS = 624
_CHUNKS = ((0, 128), (128, 128), (256, 128), (384, 128), (512, 112))
_ZR = 64
_ZCHUNKS = tuple((k * 64, 64) for k in range(9)) + ((576, 48),)


def _h0_body(x_ref, w_ref, b_ref, o_ref):
    t = jnp.dot(x_ref[...], w_ref[...], preferred_element_type=jnp.float32)
    t = t + b_ref[...]
    o_ref[...] = t * jax.nn.sigmoid(t)


def _tc_h0(x, w0, b0):
    blk = 1000
    return pl.pallas_call(
        _h0_body,
        grid=(N // blk,),
        in_specs=[
            pl.BlockSpec((blk, D), lambda i: (i, 0)),
            pl.BlockSpec((D, D), lambda i: (0, 0)),
            pl.BlockSpec((1, D), lambda i: (0, 0)),
        ],
        out_specs=pl.BlockSpec((blk, D), lambda i: (i, 0)),
        out_shape=jax.ShapeDtypeStruct((N, D), jnp.float32),
    )(x, w0, b0.reshape(1, D))


def _radial_body(dif_ref, w1_ref, b1_ref, w2_ref, b2_ref, w3_ref,
                 b3_ref, o1_ref, o2_ref, o3_ref):
    d = dif_ref[...]
    d2 = jnp.sum(d * d, axis=1, keepdims=True)
    dist = jnp.sqrt(d2 + 1e-12)
    mu = lax.broadcasted_iota(jnp.int32, (1, PPAD), 1).astype(jnp.float32) * 0.75
    sig = 6.0 / 9.0
    z = (dist - mu) / sig
    rbf = jnp.exp(-(z * z))
    for w_ref, b_ref, o_ref in ((w1_ref, b1_ref, o1_ref),
                                (w2_ref, b2_ref, o2_ref),
                                (w3_ref, b3_ref, o3_ref)):
        t = jnp.dot(rbf, w_ref[...], preferred_element_type=jnp.float32)
        t = t + b_ref[...]
        o_ref[...] = t * jax.nn.sigmoid(t)


def _tc_radial(dif, wc, bc):
    blk = 2000
    wcp = [jnp.pad(w, ((0, PPAD - 9), (0, 0))) for w in wc]
    outs = pl.pallas_call(
        _radial_body,
        grid=(E // blk,),
        in_specs=[
            pl.BlockSpec((blk, D), lambda i: (i, 0)),
        ] + [pl.BlockSpec((PPAD, D), lambda i: (0, 0)),
             pl.BlockSpec((1, D), lambda i: (0, 0))] * 3,
        out_specs=[pl.BlockSpec((blk, D), lambda i: (i, 0))] * 3,
        out_shape=[jax.ShapeDtypeStruct((E, D), jnp.float32)] * 3,
    )(dif, wcp[0], bc[0].reshape(1, D), wcp[1], bc[1].reshape(1, D),
      wcp[2], bc[2].reshape(1, D))
    return outs


def _dense_body(h_ref, a_ref, wn_ref, bn_ref, g_ref, be_ref, o_ref):
    t = h_ref[...] + a_ref[0] + a_ref[1]
    t = jnp.dot(t, wn_ref[...], preferred_element_type=jnp.float32)
    t = t + bn_ref[...]
    t = jnp.where(t >= 0, t, 0.01 * t)
    m = jnp.mean(t, axis=0, keepdims=True)
    c = t - m
    v = jnp.mean(c * c, axis=0, keepdims=True)
    o_ref[...] = c / jnp.sqrt(v + 1e-5) * g_ref[...] + be_ref[...]


def _tc_dense(h, agg, wn, bn, g, be):
    return pl.pallas_call(
        _dense_body,
        in_specs=[pl.BlockSpec((N, D), lambda: (0, 0)),
                  pl.BlockSpec((2, N, D), lambda: (0, 0, 0)),
                  pl.BlockSpec((D, D), lambda: (0, 0)),
                  pl.BlockSpec((1, D), lambda: (0, 0)),
                  pl.BlockSpec((1, D), lambda: (0, 0)),
                  pl.BlockSpec((1, D), lambda: (0, 0))],
        out_specs=pl.BlockSpec((N, D), lambda: (0, 0)),
        out_shape=jax.ShapeDtypeStruct((N, D), jnp.float32),
    )(h, agg, wn, bn.reshape(1, D), g.reshape(1, D), be.reshape(1, D))


def _graphsum_body(h_ref, b_ref, o_ref):
    onehot = (b_ref[...] == lax.broadcasted_iota(jnp.int32, (N, NUM_GRAPHS),
                                                 1)).astype(jnp.float32)
    o_ref[...] = lax.dot_general(onehot, h_ref[...], (((0,), (0,)), ((), ())),
                                 preferred_element_type=jnp.float32)


def _tc_graphsum(h, batch):
    return pl.pallas_call(
        _graphsum_body,
        in_specs=[pl.BlockSpec((N, D), lambda: (0, 0)),
                  pl.BlockSpec((N, 1), lambda: (0, 0))],
        out_specs=pl.BlockSpec((NUM_GRAPHS, D), lambda: (0, 0)),
        out_shape=jax.ShapeDtypeStruct((NUM_GRAPHS, D), jnp.float32),
    )(h, batch.reshape(N, 1))


def _ew_binop(dst, a, b, op, rows=_W):
    @pl.loop(0, rows, unroll=4)
    def _rows(r):
        for j in range(8):
            sl = (r, pl.ds(j * 16, 16))
            dst[sl] = op(a[sl], b[sl])


def _posg_body(pos_hbm, row_hbm, col_hbm, od_hbm,
               ridx_all, cidx_all, bufr0, bufr1, bufc0, bufc1, dbuf0, dbuf1,
               rs0, rs1, cs0, cs1, ws0, ws1):
    c = lax.axis_index("c")
    s = lax.axis_index("s")
    wid = s * _NC + c
    wb = wid * _NKW
    eb = wb * _W

    pltpu.sync_copy(row_hbm.at[pl.ds(eb, _NKW * _W)], ridx_all)
    pltpu.sync_copy(col_hbm.at[pl.ds(eb, _NKW * _W)], cidx_all)

    def fire(w, br, bc, rs, cs):
        idx_r = ridx_all.at[pl.ds(w * _W, _W)]
        idx_c = cidx_all.at[pl.ds(w * _W, _W)]
        pltpu.async_copy(pos_hbm.at[idx_r], br, rs)
        pltpu.async_copy(pos_hbm.at[idx_c], bc, cs)

    def wait_g(br, bc, rs, cs):
        pltpu.make_async_copy(pos_hbm.at[ridx_all.at[pl.ds(0, _W)]], br,
                              rs).wait()
        pltpu.make_async_copy(pos_hbm.at[ridx_all.at[pl.ds(0, _W)]], bc,
                              cs).wait()

    def wait_w(db, ws):
        pltpu.make_async_copy(db, od_hbm.at[pl.ds(0, _W), :], ws).wait()

    fire(0, bufr0, bufc0, rs0, cs0)

    @pl.loop(0, _PAIRS)
    def _pair(p):
        w0 = 2 * p
        fire(w0 + 1, bufr1, bufc1, rs1, cs1)
        wait_g(bufr0, bufc0, rs0, cs0)

        @pl.when(p > 0)
        def _():
            wait_w(dbuf0, ws0)

        _ew_binop(dbuf0, bufr0, bufc0, lax.sub)
        pltpu.async_copy(dbuf0, od_hbm.at[pl.ds((wb + w0) * _W, _W), :], ws0)

        @pl.when(p < _PAIRS - 1)
        def _():
            fire(w0 + 2, bufr0, bufc0, rs0, cs0)

        wait_g(bufr1, bufc1, rs1, cs1)

        @pl.when(p > 0)
        def _():
            wait_w(dbuf1, ws1)

        _ew_binop(dbuf1, bufr1, bufc1, lax.sub)
        pltpu.async_copy(dbuf1, od_hbm.at[pl.ds((wb + w0 + 1) * _W, _W), :],
                         ws1)

    wait_w(dbuf0, ws0)
    wait_w(dbuf1, ws1)

    @pl.when(wid < _NLEFT)
    def _left():
        wi = _NKW * _NW + wid
        pltpu.sync_copy(row_hbm.at[pl.ds(wi * _W, _W)],
                        ridx_all.at[pl.ds(0, _W)])
        pltpu.sync_copy(col_hbm.at[pl.ds(wi * _W, _W)],
                        cidx_all.at[pl.ds(0, _W)])
        fire(0, bufr0, bufc0, rs0, cs0)
        wait_g(bufr0, bufc0, rs0, cs0)
        _ew_binop(dbuf0, bufr0, bufc0, lax.sub)
        pltpu.sync_copy(dbuf0, od_hbm.at[pl.ds(wi * _W, _W), :])


def _sc_pos_diff(pos128, row, col):
    mesh = plsc.VectorSubcoreMesh(core_axis_name="c", subcore_axis_name="s")
    f = pl.kernel(
        _posg_body,
        out_type=jax.ShapeDtypeStruct((E, D), jnp.float32),
        mesh=mesh,
        scratch_types=[
            pltpu.VMEM((_NKW * _W,), jnp.int32),
            pltpu.VMEM((_NKW * _W,), jnp.int32),
        ] + [pltpu.VMEM((_W, D), jnp.float32)] * 6
          + [pltpu.SemaphoreType.DMA] * 6,
    )
    return f(pos128, row, col)


def _edge_body(h_hbm, rad_hbm, row_hbm, col_hbm, out_hbm,
               aggs, ridx0, ridx1, ridx2, ridx3, cidx0, cidx1, cidx2, cidx3,
               gath0, gath1, rad0, rad1,
               gs0, gs1, rs0, rs1, is0, is1, is2, is3, js0, js1, js2, js3):
    c = lax.axis_index("c")
    s = lax.axis_index("s")
    wid = s * _NC + c
    wb = wid * _NKWE

    z16 = jnp.zeros((16,), jnp.float32)

    @pl.loop(0, _WE)
    def _zb(r):
        for j in range(8):
            gath0[r, pl.ds(j * 16, 16)] = z16

    base = s * _RPS
    for off, sz in _ZCHUNKS:
        pltpu.sync_copy(gath0.at[pl.ds(0, sz), :],
                        aggs.at[pl.ds(base + off, sz), :])

    @pl.when(s == _NS - 1)
    def _ztail():
        pltpu.sync_copy(gath0.at[pl.ds(0, 16), :],
                        aggs.at[pl.ds(_NS * _RPS, 16), :])

    plsc.subcore_barrier()

    def fire_idx(w, ridx, cidx, isem, jsem):
        pltpu.async_copy(row_hbm.at[pl.ds((wb + w) * _WE, _WE)], ridx, isem)
        pltpu.async_copy(col_hbm.at[pl.ds((wb + w) * _WE, _WE)], cidx, jsem)

    def fire_data(w, ridx, gath, rad, isem, gs, rs):
        pltpu.make_async_copy(row_hbm.at[pl.ds(0, _WE)], ridx, isem).wait()
        pltpu.async_copy(h_hbm.at[ridx], gath, gs)
        pltpu.async_copy(rad_hbm.at[pl.ds((wb + w) * _WE, _WE), :], rad, rs)

    def wait_in(ridx, gath, rad, cidx, gs, rs, jsem):
        pltpu.make_async_copy(col_hbm.at[pl.ds(0, _WE)], cidx, jsem).wait()
        pltpu.make_async_copy(rad_hbm.at[pl.ds(0, _WE), :], rad, rs).wait()
        pltpu.make_async_copy(h_hbm.at[ridx], gath, gs).wait()

    ridx = (ridx0, ridx1, ridx2, ridx3)
    cidx = (cidx0, cidx1, cidx2, cidx3)
    isem = (is0, is1, is2, is3)
    jsem = (js0, js1, js2, js3)
    gath = (gath0, gath1)
    rad = (rad0, rad1)
    gsem = (gs0, gs1)
    rsem = (rs0, rs1)

    for q in range(4):
        fire_idx(q, ridx[q], cidx[q], isem[q], jsem[q])
    for b in range(2):
        fire_data(b, ridx[b], gath[b], rad[b], isem[b], gsem[b], rsem[b])

    @pl.loop(0, _QUADS)
    def _quad(p):
        w0 = 4 * p
        for o in range(4):
            b = o % 2
            q = o
            qn = (o + 2) % 4
            wait_in(ridx[q], gath[b], rad[b], cidx[q],
                    gsem[b], rsem[b], jsem[q])
            _ew_binop(gath[b], gath[b], rad[b], lax.mul, rows=_WE)
            pltpu.sync_copy(gath[b], aggs.at[cidx[q]], add=True)
            @pl.when(p < _QUADS - 1)
            def _(o=o, q=q, w0=w0):
                pltpu.async_copy(
                    row_hbm.at[pl.ds((wb + w0 + o + 4) * _WE, _WE)],
                    ridx[q], isem[q])
                pltpu.async_copy(
                    col_hbm.at[pl.ds((wb + w0 + o + 4) * _WE, _WE)],
                    cidx[q], jsem[q])

            if o < 2:
                fire_data(w0 + o + 2, ridx[qn], gath[b], rad[b],
                          isem[qn], gsem[b], rsem[b])
            else:
                @pl.when(p < _QUADS - 1)
                def _(o=o, b=b, qn=qn, w0=w0):
                    fire_data(w0 + o + 2, ridx[qn], gath[b], rad[b],
                              isem[qn], gsem[b], rsem[b])

    @pl.when(wid < _NLEFTE)
    def _left():
        wi = _NKWE * _NW + wid
        pltpu.sync_copy(row_hbm.at[pl.ds(wi * _WE, _WE)], ridx0)
        pltpu.sync_copy(col_hbm.at[pl.ds(wi * _WE, _WE)], cidx0)
        pltpu.async_copy(h_hbm.at[ridx0], gath0, gs0).wait()
        pltpu.sync_copy(rad_hbm.at[pl.ds(wi * _WE, _WE), :], rad0)
        _ew_binop(gath0, gath0, rad0, lax.mul, rows=_WE)
        pltpu.sync_copy(gath0, aggs.at[cidx0], add=True)

    plsc.subcore_barrier()
    for off, sz in _ZCHUNKS:
        pltpu.sync_copy(aggs.at[pl.ds(base + off, sz), :],
                        out_hbm.at[c, pl.ds(base + off, sz), :])

    @pl.when(s == _NS - 1)
    def _wtail():
        pltpu.sync_copy(aggs.at[pl.ds(_NS * _RPS, 16), :],
                        out_hbm.at[c, pl.ds(_NS * _RPS, 16), :])


def _sc_edge_pass(h, radial, row, col):
    mesh = plsc.VectorSubcoreMesh(core_axis_name="c", subcore_axis_name="s")
    f = pl.kernel(
        _edge_body,
        out_type=jax.ShapeDtypeStruct((_NC, N, D), jnp.float32),
        mesh=mesh,
        scratch_types=[
            pltpu.VMEM_SHARED((N, D), jnp.float32),
        ] + [pltpu.VMEM((_WE,), jnp.int32)] * 8
          + [pltpu.VMEM((_WE, D), jnp.float32)] * 4
          + [pltpu.SemaphoreType.DMA] * 12,
    )
    return f(h, radial, row, col)


def kernel(x, edge_index, pos, edge_attr, batch, W0, b0,
           Wc1, bc1, Wn1, bn1, g1, be1,
           Wc2, bc2, Wn2, bn2, g2, be2,
           Wc3, bc3, Wn3, bn3, g3, be3):
    row = edge_index[0].astype(jnp.int32)
    col = edge_index[1].astype(jnp.int32)
    pos128 = jnp.pad(pos, ((0, 0), (0, D - 3)))
    dif = _sc_pos_diff(pos128, row, col)
    h = _tc_h0(x, W0, b0)
    r1, r2, r3 = _tc_radial(dif, (Wc1, Wc2, Wc3), (bc1, bc2, bc3))
    for radial, wn, bn, g, be in ((r1, Wn1, bn1, g1, be1),
                                  (r2, Wn2, bn2, g2, be2),
                                  (r3, Wn3, bn3, g3, be3)):
        agg = _sc_edge_pass(h, radial, row, col)
        h = _tc_dense(h, agg, wn, bn, g, be)
    return _tc_graphsum(h, batch)


# speedup vs baseline: 1.1251x; 1.1251x over previous
import functools

import jax
import jax.numpy as jnp
from jax import lax
from jax.experimental import pallas as pl
from jax.experimental.pallas import tpu as pltpu
from jax.experimental.pallas import tpu_sc as plsc

N = 10000
E = 320000
D = 128
NUM_GRAPHS = 64
PPAD = 16

_NC, _NS = 2, 16
_NW = _NC * _NS
_W = 128
_NWIN = E // _W
_NKW = 2496 // _NW
_PAIRS = _NKW // 2
_NLEFT = _NWIN - _NKW * _NW
_WE = 64
_NWINE = E // _WE
_NKWE = (_NWINE // _NW) & ~3
_QUADS = _NKWE // 4
_NLEFTE = _NWINE - _NKWE * _NW
_RPS = 624
_CHUNKS = ((0, 128), (128, 128), (256, 128), (384, 128), (512, 112))
_ZR = 64
_ZCHUNKS = tuple((k * 64, 64) for k in range(9)) + ((576, 48),)


def _h0_body(x_ref, w_ref, b_ref, o_ref):
    t = jnp.dot(x_ref[...], w_ref[...], preferred_element_type=jnp.float32)
    t = t + b_ref[...]
    o_ref[...] = t * jax.nn.sigmoid(t)


def _tc_h0(x, w0, b0):
    blk = 1000
    return pl.pallas_call(
        _h0_body,
        grid=(N // blk,),
        in_specs=[
            pl.BlockSpec((blk, D), lambda i: (i, 0)),
            pl.BlockSpec((D, D), lambda i: (0, 0)),
            pl.BlockSpec((1, D), lambda i: (0, 0)),
        ],
        out_specs=pl.BlockSpec((blk, D), lambda i: (i, 0)),
        out_shape=jax.ShapeDtypeStruct((N, D), jnp.float32),
    )(x, w0, b0.reshape(1, D))


def _radial_body(dif_ref, w1_ref, b1_ref, w2_ref, b2_ref, w3_ref,
                 b3_ref, o1_ref, o2_ref, o3_ref):
    d = dif_ref[...]
    d2 = jnp.sum(d * d, axis=1, keepdims=True)
    dist = jnp.sqrt(d2 + 1e-12)
    mu = lax.broadcasted_iota(jnp.int32, (1, PPAD), 1).astype(jnp.float32) * 0.75
    sig = 6.0 / 9.0
    z = (dist - mu) / sig
    rbf = jnp.exp(-(z * z))
    for w_ref, b_ref, o_ref in ((w1_ref, b1_ref, o1_ref),
                                (w2_ref, b2_ref, o2_ref),
                                (w3_ref, b3_ref, o3_ref)):
        t = jnp.dot(rbf, w_ref[...], preferred_element_type=jnp.float32)
        t = t + b_ref[...]
        o_ref[...] = t * jax.nn.sigmoid(t)


def _tc_radial(dif, wc, bc):
    blk = 2000
    wcp = [jnp.pad(w, ((0, PPAD - 9), (0, 0))) for w in wc]
    outs = pl.pallas_call(
        _radial_body,
        grid=(E // blk,),
        in_specs=[
            pl.BlockSpec((blk, PPAD), lambda i: (i, 0)),
        ] + [pl.BlockSpec((PPAD, D), lambda i: (0, 0)),
             pl.BlockSpec((1, D), lambda i: (0, 0))] * 3,
        out_specs=[pl.BlockSpec((blk, D), lambda i: (i, 0))] * 3,
        out_shape=[jax.ShapeDtypeStruct((E, D), jnp.float32)] * 3,
    )(dif, wcp[0], bc[0].reshape(1, D), wcp[1], bc[1].reshape(1, D),
      wcp[2], bc[2].reshape(1, D))
    return outs


def _dense_body(h_ref, a_ref, wn_ref, bn_ref, g_ref, be_ref, o_ref):
    t = h_ref[...] + a_ref[0] + a_ref[1]
    t = jnp.dot(t, wn_ref[...], preferred_element_type=jnp.float32)
    t = t + bn_ref[...]
    t = jnp.where(t >= 0, t, 0.01 * t)
    m = jnp.mean(t, axis=0, keepdims=True)
    c = t - m
    v = jnp.mean(c * c, axis=0, keepdims=True)
    o_ref[...] = c / jnp.sqrt(v + 1e-5) * g_ref[...] + be_ref[...]


def _tc_dense(h, agg, wn, bn, g, be):
    return pl.pallas_call(
        _dense_body,
        in_specs=[pl.BlockSpec((N, D), lambda: (0, 0)),
                  pl.BlockSpec((2, N, D), lambda: (0, 0, 0)),
                  pl.BlockSpec((D, D), lambda: (0, 0)),
                  pl.BlockSpec((1, D), lambda: (0, 0)),
                  pl.BlockSpec((1, D), lambda: (0, 0)),
                  pl.BlockSpec((1, D), lambda: (0, 0))],
        out_specs=pl.BlockSpec((N, D), lambda: (0, 0)),
        out_shape=jax.ShapeDtypeStruct((N, D), jnp.float32),
    )(h, agg, wn, bn.reshape(1, D), g.reshape(1, D), be.reshape(1, D))


def _graphsum_body(h_ref, b_ref, o_ref):
    onehot = (b_ref[...] == lax.broadcasted_iota(jnp.int32, (N, NUM_GRAPHS),
                                                 1)).astype(jnp.float32)
    o_ref[...] = lax.dot_general(onehot, h_ref[...], (((0,), (0,)), ((), ())),
                                 preferred_element_type=jnp.float32)


def _tc_graphsum(h, batch):
    return pl.pallas_call(
        _graphsum_body,
        in_specs=[pl.BlockSpec((N, D), lambda: (0, 0)),
                  pl.BlockSpec((N, 1), lambda: (0, 0))],
        out_specs=pl.BlockSpec((NUM_GRAPHS, D), lambda: (0, 0)),
        out_shape=jax.ShapeDtypeStruct((NUM_GRAPHS, D), jnp.float32),
    )(h, batch.reshape(N, 1))


def _ew_binop(dst, a, b, op, rows=_W, cols=D):
    @pl.loop(0, rows, unroll=4)
    def _rows(r):
        for j in range(cols // 16):
            sl = (r, pl.ds(j * 16, 16))
            dst[sl] = op(a[sl], b[sl])


def _posg_body(pos_hbm, row_hbm, col_hbm, od_hbm,
               ridx_all, cidx_all, bufr0, bufr1, bufc0, bufc1, dbuf0, dbuf1,
               rs0, rs1, cs0, cs1, ws0, ws1):
    c = lax.axis_index("c")
    s = lax.axis_index("s")
    wid = s * _NC + c
    wb = wid * _NKW
    eb = wb * _W

    pltpu.sync_copy(row_hbm.at[pl.ds(eb, _NKW * _W)], ridx_all)
    pltpu.sync_copy(col_hbm.at[pl.ds(eb, _NKW * _W)], cidx_all)

    def fire(w, br, bc, rs, cs):
        idx_r = ridx_all.at[pl.ds(w * _W, _W)]
        idx_c = cidx_all.at[pl.ds(w * _W, _W)]
        pltpu.async_copy(pos_hbm.at[idx_r], br, rs)
        pltpu.async_copy(pos_hbm.at[idx_c], bc, cs)

    def wait_g(br, bc, rs, cs):
        pltpu.make_async_copy(pos_hbm.at[ridx_all.at[pl.ds(0, _W)]], br,
                              rs).wait()
        pltpu.make_async_copy(pos_hbm.at[ridx_all.at[pl.ds(0, _W)]], bc,
                              cs).wait()

    def wait_w(db, ws):
        pltpu.make_async_copy(db, od_hbm.at[pl.ds(0, _W), :], ws).wait()

    fire(0, bufr0, bufc0, rs0, cs0)

    @pl.loop(0, _PAIRS)
    def _pair(p):
        w0 = 2 * p
        fire(w0 + 1, bufr1, bufc1, rs1, cs1)
        wait_g(bufr0, bufc0, rs0, cs0)

        @pl.when(p > 0)
        def _():
            wait_w(dbuf0, ws0)

        _ew_binop(dbuf0, bufr0, bufc0, lax.sub, cols=PPAD)
        pltpu.async_copy(dbuf0, od_hbm.at[pl.ds((wb + w0) * _W, _W), :], ws0)

        @pl.when(p < _PAIRS - 1)
        def _():
            fire(w0 + 2, bufr0, bufc0, rs0, cs0)

        wait_g(bufr1, bufc1, rs1, cs1)

        @pl.when(p > 0)
        def _():
            wait_w(dbuf1, ws1)

        _ew_binop(dbuf1, bufr1, bufc1, lax.sub, cols=PPAD)
        pltpu.async_copy(dbuf1, od_hbm.at[pl.ds((wb + w0 + 1) * _W, _W), :],
                         ws1)

    wait_w(dbuf0, ws0)
    wait_w(dbuf1, ws1)

    @pl.when(wid < _NLEFT)
    def _left():
        wi = _NKW * _NW + wid
        pltpu.sync_copy(row_hbm.at[pl.ds(wi * _W, _W)],
                        ridx_all.at[pl.ds(0, _W)])
        pltpu.sync_copy(col_hbm.at[pl.ds(wi * _W, _W)],
                        cidx_all.at[pl.ds(0, _W)])
        fire(0, bufr0, bufc0, rs0, cs0)
        wait_g(bufr0, bufc0, rs0, cs0)
        _ew_binop(dbuf0, bufr0, bufc0, lax.sub, cols=PPAD)
        pltpu.sync_copy(dbuf0, od_hbm.at[pl.ds(wi * _W, _W), :])


def _sc_pos_diff(pos128, row, col):
    mesh = plsc.VectorSubcoreMesh(core_axis_name="c", subcore_axis_name="s")
    f = pl.kernel(
        _posg_body,
        out_type=jax.ShapeDtypeStruct((E, PPAD), jnp.float32),
        mesh=mesh,
        scratch_types=[
            pltpu.VMEM((_NKW * _W,), jnp.int32),
            pltpu.VMEM((_NKW * _W,), jnp.int32),
        ] + [pltpu.VMEM((_W, D), jnp.float32)] * 4
          + [pltpu.VMEM((_W, PPAD), jnp.float32)] * 2
          + [pltpu.SemaphoreType.DMA] * 6,
    )
    return f(pos128, row, col)


def _edge_body(h_hbm, rad_hbm, row_hbm, col_hbm, out_hbm,
               aggs, ridx0, ridx1, ridx2, ridx3, cidx0, cidx1, cidx2, cidx3,
               gath0, gath1, rad0, rad1,
               gs0, gs1, rs0, rs1, is0, is1, is2, is3, js0, js1, js2, js3):
    c = lax.axis_index("c")
    s = lax.axis_index("s")
    wid = s * _NC + c
    wb = wid * _NKWE

    z16 = jnp.zeros((16,), jnp.float32)

    @pl.loop(0, _WE)
    def _zb(r):
        for j in range(8):
            gath0[r, pl.ds(j * 16, 16)] = z16

    base = s * _RPS
    for off, sz in _ZCHUNKS:
        pltpu.sync_copy(gath0.at[pl.ds(0, sz), :],
                        aggs.at[pl.ds(base + off, sz), :])

    @pl.when(s == _NS - 1)
    def _ztail():
        pltpu.sync_copy(gath0.at[pl.ds(0, 16), :],
                        aggs.at[pl.ds(_NS * _RPS, 16), :])

    plsc.subcore_barrier()

    def fire_idx(w, ridx, cidx, isem, jsem):
        pltpu.async_copy(row_hbm.at[pl.ds((wb + w) * _WE, _WE)], ridx, isem)
        pltpu.async_copy(col_hbm.at[pl.ds((wb + w) * _WE, _WE)], cidx, jsem)

    def fire_data(w, ridx, gath, rad, isem, gs, rs):
        pltpu.make_async_copy(row_hbm.at[pl.ds(0, _WE)], ridx, isem).wait()
        pltpu.async_copy(h_hbm.at[ridx], gath, gs)
        pltpu.async_copy(rad_hbm.at[pl.ds((wb + w) * _WE, _WE), :], rad, rs)

    def wait_in(ridx, gath, rad, cidx, gs, rs, jsem):
        pltpu.make_async_copy(col_hbm.at[pl.ds(0, _WE)], cidx, jsem).wait()
        pltpu.make_async_copy(rad_hbm.at[pl.ds(0, _WE), :], rad, rs).wait()
        pltpu.make_async_copy(h_hbm.at[ridx], gath, gs).wait()

    ridx = (ridx0, ridx1, ridx2, ridx3)
    cidx = (cidx0, cidx1, cidx2, cidx3)
    isem = (is0, is1, is2, is3)
    jsem = (js0, js1, js2, js3)
    gath = (gath0, gath1)
    rad = (rad0, rad1)
    gsem = (gs0, gs1)
    rsem = (rs0, rs1)

    for q in range(4):
        fire_idx(q, ridx[q], cidx[q], isem[q], jsem[q])
    for b in range(2):
        fire_data(b, ridx[b], gath[b], rad[b], isem[b], gsem[b], rsem[b])

    @pl.loop(0, _QUADS)
    def _quad(p):
        w0 = 4 * p
        for o in range(4):
            b = o % 2
            q = o
            qn = (o + 2) % 4
            wait_in(ridx[q], gath[b], rad[b], cidx[q],
                    gsem[b], rsem[b], jsem[q])
            _ew_binop(gath[b], gath[b], rad[b], lax.mul, rows=_WE)
            pltpu.sync_copy(gath[b], aggs.at[cidx[q]], add=True)
            @pl.when(p < _QUADS - 1)
            def _(o=o, q=q, w0=w0):
                pltpu.async_copy(
                    row_hbm.at[pl.ds((wb + w0 + o + 4) * _WE, _WE)],
                    ridx[q], isem[q])
                pltpu.async_copy(
                    col_hbm.at[pl.ds((wb + w0 + o + 4) * _WE, _WE)],
                    cidx[q], jsem[q])

            if o < 2:
                fire_data(w0 + o + 2, ridx[qn], gath[b], rad[b],
                          isem[qn], gsem[b], rsem[b])
            else:
                @pl.when(p < _QUADS - 1)
                def _(o=o, b=b, qn=qn, w0=w0):
                    fire_data(w0 + o + 2, ridx[qn], gath[b], rad[b],
                              isem[qn], gsem[b], rsem[b])

    @pl.when(wid < _NLEFTE)
    def _left():
        wi = _NKWE * _NW + wid
        pltpu.sync_copy(row_hbm.at[pl.ds(wi * _WE, _WE)], ridx0)
        pltpu.sync_copy(col_hbm.at[pl.ds(wi * _WE, _WE)], cidx0)
        pltpu.async_copy(h_hbm.at[ridx0], gath0, gs0).wait()
        pltpu.sync_copy(rad_hbm.at[pl.ds(wi * _WE, _WE), :], rad0)
        _ew_binop(gath0, gath0, rad0, lax.mul, rows=_WE)
        pltpu.sync_copy(gath0, aggs.at[cidx0], add=True)

    plsc.subcore_barrier()
    for off, sz in _ZCHUNKS:
        pltpu.sync_copy(aggs.at[pl.ds(base + off, sz), :],
                        out_hbm.at[c, pl.ds(base + off, sz), :])

    @pl.when(s == _NS - 1)
    def _wtail():
        pltpu.sync_copy(aggs.at[pl.ds(_NS * _RPS, 16), :],
                        out_hbm.at[c, pl.ds(_NS * _RPS, 16), :])


def _sc_edge_pass(h, radial, row, col):
    mesh = plsc.VectorSubcoreMesh(core_axis_name="c", subcore_axis_name="s")
    f = pl.kernel(
        _edge_body,
        out_type=jax.ShapeDtypeStruct((_NC, N, D), jnp.float32),
        mesh=mesh,
        scratch_types=[
            pltpu.VMEM_SHARED((N, D), jnp.float32),
        ] + [pltpu.VMEM((_WE,), jnp.int32)] * 8
          + [pltpu.VMEM((_WE, D), jnp.float32)] * 4
          + [pltpu.SemaphoreType.DMA] * 12,
    )
    return f(h, radial, row, col)


def kernel(x, edge_index, pos, edge_attr, batch, W0, b0,
           Wc1, bc1, Wn1, bn1, g1, be1,
           Wc2, bc2, Wn2, bn2, g2, be2,
           Wc3, bc3, Wn3, bn3, g3, be3):
    row = edge_index[0].astype(jnp.int32)
    col = edge_index[1].astype(jnp.int32)
    pos128 = jnp.pad(pos, ((0, 0), (0, D - 3)))
    dif = _sc_pos_diff(pos128, row, col)
    h = _tc_h0(x, W0, b0)
    r1, r2, r3 = _tc_radial(dif, (Wc1, Wc2, Wc3), (bc1, bc2, bc3))
    for radial, wn, bn, g, be in ((r1, Wn1, bn1, g1, be1),
                                  (r2, Wn2, bn2, g2, be2),
                                  (r3, Wn3, bn3, g3, be3)):
        agg = _sc_edge_pass(h, radial, row, col)
        h = _tc_dense(h, agg, wn, bn, g, be)
    return _tc_graphsum(h, batch)


# speedup vs baseline: 1.1786x; 1.0475x over previous
import functools

import jax
import jax.numpy as jnp
from jax import lax
from jax.experimental import pallas as pl
from jax.experimental.pallas import tpu as pltpu
from jax.experimental.pallas import tpu_sc as plsc

N = 10000
E = 320000
D = 128
NUM_GRAPHS = 64
PPAD = 16

_NC, _NS = 2, 16
_NW = _NC * _NS
_W = 128
_NWIN = E // _W
_NKW = 2496 // _NW
_PAIRS = _NKW // 2
_NLEFT = _NWIN - _NKW * _NW
_WE = 64
_NWINE = E // _WE
_NKWE = (_NWINE // _NW) & ~3
_QUADS = _NKWE // 4
_NLEFTE = _NWINE - _NKWE * _NW
_RPS = 624
_CHUNKS = ((0, 128), (128, 128), (256, 128), (384, 128), (512, 112))
_ZR = 64
_ZCHUNKS = tuple((k * 64, 64) for k in range(9)) + ((576, 48),)


def _h0_body(x_ref, w_ref, b_ref, o_ref):
    t = jnp.dot(x_ref[...], w_ref[...], preferred_element_type=jnp.float32)
    t = t + b_ref[...]
    o_ref[...] = t * jax.nn.sigmoid(t)


def _tc_h0(x, w0, b0):
    blk = 1000
    return pl.pallas_call(
        _h0_body,
        grid=(N // blk,),
        in_specs=[
            pl.BlockSpec((blk, D), lambda i: (i, 0)),
            pl.BlockSpec((D, D), lambda i: (0, 0)),
            pl.BlockSpec((1, D), lambda i: (0, 0)),
        ],
        out_specs=pl.BlockSpec((blk, D), lambda i: (i, 0)),
        out_shape=jax.ShapeDtypeStruct((N, D), jnp.float32),
    )(x, w0, b0.reshape(1, D))


def _radial_body(dif_ref, w1_ref, b1_ref, w2_ref, b2_ref, w3_ref,
                 b3_ref, o1_ref, o2_ref, o3_ref):
    d = dif_ref[...]
    d2 = jnp.sum(d * d, axis=1, keepdims=True)
    dist = jnp.sqrt(d2 + 1e-12)
    mu = lax.broadcasted_iota(jnp.int32, (1, PPAD), 1).astype(jnp.float32) * 0.75
    sig = 6.0 / 9.0
    z = (dist - mu) / sig
    rbf = jnp.exp(-(z * z))
    for w_ref, b_ref, o_ref in ((w1_ref, b1_ref, o1_ref),
                                (w2_ref, b2_ref, o2_ref),
                                (w3_ref, b3_ref, o3_ref)):
        t = jnp.dot(rbf, w_ref[...], preferred_element_type=jnp.float32)
        t = t + b_ref[...]
        o_ref[...] = (t * jax.nn.sigmoid(t)).astype(jnp.bfloat16)


def _tc_radial(dif, wc, bc):
    blk = 2000
    wcp = [jnp.pad(w, ((0, PPAD - 9), (0, 0))) for w in wc]
    outs = pl.pallas_call(
        _radial_body,
        grid=(E // blk,),
        in_specs=[
            pl.BlockSpec((blk, PPAD), lambda i: (i, 0)),
        ] + [pl.BlockSpec((PPAD, D), lambda i: (0, 0)),
             pl.BlockSpec((1, D), lambda i: (0, 0))] * 3,
        out_specs=[pl.BlockSpec((blk, D), lambda i: (i, 0))] * 3,
        out_shape=[jax.ShapeDtypeStruct((E, D), jnp.bfloat16)] * 3,
    )(dif, wcp[0], bc[0].reshape(1, D), wcp[1], bc[1].reshape(1, D),
      wcp[2], bc[2].reshape(1, D))
    return outs


def _dense_body(h_ref, a_ref, wn_ref, bn_ref, g_ref, be_ref, o_ref):
    t = h_ref[...] + a_ref[0] + a_ref[1]
    t = jnp.dot(t, wn_ref[...], preferred_element_type=jnp.float32)
    t = t + bn_ref[...]
    t = jnp.where(t >= 0, t, 0.01 * t)
    m = jnp.mean(t, axis=0, keepdims=True)
    c = t - m
    v = jnp.mean(c * c, axis=0, keepdims=True)
    o_ref[...] = c / jnp.sqrt(v + 1e-5) * g_ref[...] + be_ref[...]


def _tc_dense(h, agg, wn, bn, g, be):
    return pl.pallas_call(
        _dense_body,
        in_specs=[pl.BlockSpec((N, D), lambda: (0, 0)),
                  pl.BlockSpec((2, N, D), lambda: (0, 0, 0)),
                  pl.BlockSpec((D, D), lambda: (0, 0)),
                  pl.BlockSpec((1, D), lambda: (0, 0)),
                  pl.BlockSpec((1, D), lambda: (0, 0)),
                  pl.BlockSpec((1, D), lambda: (0, 0))],
        out_specs=pl.BlockSpec((N, D), lambda: (0, 0)),
        out_shape=jax.ShapeDtypeStruct((N, D), jnp.float32),
    )(h, agg, wn, bn.reshape(1, D), g.reshape(1, D), be.reshape(1, D))


def _graphsum_body(h_ref, b_ref, o_ref):
    onehot = (b_ref[...] == lax.broadcasted_iota(jnp.int32, (N, NUM_GRAPHS),
                                                 1)).astype(jnp.float32)
    o_ref[...] = lax.dot_general(onehot, h_ref[...], (((0,), (0,)), ((), ())),
                                 preferred_element_type=jnp.float32)


def _tc_graphsum(h, batch):
    return pl.pallas_call(
        _graphsum_body,
        in_specs=[pl.BlockSpec((N, D), lambda: (0, 0)),
                  pl.BlockSpec((N, 1), lambda: (0, 0))],
        out_specs=pl.BlockSpec((NUM_GRAPHS, D), lambda: (0, 0)),
        out_shape=jax.ShapeDtypeStruct((NUM_GRAPHS, D), jnp.float32),
    )(h, batch.reshape(N, 1))


def _ew_binop(dst, a, b, op, rows=_W, cols=D, static=False):
    if static:
        for r in range(rows):
            for j in range(cols // 16):
                sl = (r, pl.ds(j * 16, 16))
                dst[sl] = op(a[sl], b[sl])
        return

    @pl.loop(0, rows, unroll=4)
    def _rows(r):
        for j in range(cols // 16):
            sl = (r, pl.ds(j * 16, 16))
            dst[sl] = op(a[sl], b[sl])


def _posg_body(pos_hbm, row_hbm, col_hbm, od_hbm,
               ridx_all, cidx_all, bufr0, bufr1, bufc0, bufc1, dbuf0, dbuf1,
               rs0, rs1, cs0, cs1, ws0, ws1):
    c = lax.axis_index("c")
    s = lax.axis_index("s")
    wid = s * _NC + c
    wb = wid * _NKW
    eb = wb * _W

    pltpu.sync_copy(row_hbm.at[pl.ds(eb, _NKW * _W)], ridx_all)
    pltpu.sync_copy(col_hbm.at[pl.ds(eb, _NKW * _W)], cidx_all)

    def fire(w, br, bc, rs, cs):
        idx_r = ridx_all.at[pl.ds(w * _W, _W)]
        idx_c = cidx_all.at[pl.ds(w * _W, _W)]
        pltpu.async_copy(pos_hbm.at[idx_r], br, rs)
        pltpu.async_copy(pos_hbm.at[idx_c], bc, cs)

    def wait_g(br, bc, rs, cs):
        pltpu.make_async_copy(pos_hbm.at[ridx_all.at[pl.ds(0, _W)]], br,
                              rs).wait()
        pltpu.make_async_copy(pos_hbm.at[ridx_all.at[pl.ds(0, _W)]], bc,
                              cs).wait()

    def wait_w(db, ws):
        pltpu.make_async_copy(db, od_hbm.at[pl.ds(0, _W), :], ws).wait()

    fire(0, bufr0, bufc0, rs0, cs0)

    @pl.loop(0, _PAIRS)
    def _pair(p):
        w0 = 2 * p
        fire(w0 + 1, bufr1, bufc1, rs1, cs1)
        wait_g(bufr0, bufc0, rs0, cs0)

        @pl.when(p > 0)
        def _():
            wait_w(dbuf0, ws0)

        _ew_binop(dbuf0, bufr0, bufc0, lax.sub, cols=PPAD)
        pltpu.async_copy(dbuf0, od_hbm.at[pl.ds((wb + w0) * _W, _W), :], ws0)

        @pl.when(p < _PAIRS - 1)
        def _():
            fire(w0 + 2, bufr0, bufc0, rs0, cs0)

        wait_g(bufr1, bufc1, rs1, cs1)

        @pl.when(p > 0)
        def _():
            wait_w(dbuf1, ws1)

        _ew_binop(dbuf1, bufr1, bufc1, lax.sub, cols=PPAD)
        pltpu.async_copy(dbuf1, od_hbm.at[pl.ds((wb + w0 + 1) * _W, _W), :],
                         ws1)

    wait_w(dbuf0, ws0)
    wait_w(dbuf1, ws1)

    @pl.when(wid < _NLEFT)
    def _left():
        wi = _NKW * _NW + wid
        pltpu.sync_copy(row_hbm.at[pl.ds(wi * _W, _W)],
                        ridx_all.at[pl.ds(0, _W)])
        pltpu.sync_copy(col_hbm.at[pl.ds(wi * _W, _W)],
                        cidx_all.at[pl.ds(0, _W)])
        fire(0, bufr0, bufc0, rs0, cs0)
        wait_g(bufr0, bufc0, rs0, cs0)
        _ew_binop(dbuf0, bufr0, bufc0, lax.sub, cols=PPAD)
        pltpu.sync_copy(dbuf0, od_hbm.at[pl.ds(wi * _W, _W), :])


def _sc_pos_diff(pos128, row, col):
    mesh = plsc.VectorSubcoreMesh(core_axis_name="c", subcore_axis_name="s")
    f = pl.kernel(
        _posg_body,
        out_type=jax.ShapeDtypeStruct((E, PPAD), jnp.float32),
        mesh=mesh,
        scratch_types=[
            pltpu.VMEM((_NKW * _W,), jnp.int32),
            pltpu.VMEM((_NKW * _W,), jnp.int32),
        ] + [pltpu.VMEM((_W, D), jnp.float32)] * 4
          + [pltpu.VMEM((_W, PPAD), jnp.float32)] * 2
          + [pltpu.SemaphoreType.DMA] * 6,
    )
    return f(pos128, row, col)


def _edge_body(h_hbm, rad_hbm, row_hbm, col_hbm, out_hbm,
               aggs, ridx0, ridx1, ridx2, ridx3, cidx0, cidx1, cidx2, cidx3,
               gath0, gath1, rad0, rad1,
               gs0, gs1, rs0, rs1, is0, is1, is2, is3, js0, js1, js2, js3):
    c = lax.axis_index("c")
    s = lax.axis_index("s")
    wid = s * _NC + c
    wb = wid * _NKWE

    z16 = jnp.zeros((16,), jnp.float32)

    def mul_bf(x, y):
        return x * y.astype(jnp.float32)

    @pl.loop(0, _WE)
    def _zb(r):
        for j in range(8):
            gath0[r, pl.ds(j * 16, 16)] = z16

    base = s * _RPS
    for off, sz in _ZCHUNKS:
        pltpu.sync_copy(gath0.at[pl.ds(0, sz), :],
                        aggs.at[pl.ds(base + off, sz), :])

    @pl.when(s == _NS - 1)
    def _ztail():
        pltpu.sync_copy(gath0.at[pl.ds(0, 16), :],
                        aggs.at[pl.ds(_NS * _RPS, 16), :])

    plsc.subcore_barrier()

    def fire_idx(w, ridx, cidx, isem, jsem):
        pltpu.async_copy(row_hbm.at[pl.ds((wb + w) * _WE, _WE)], ridx, isem)
        pltpu.async_copy(col_hbm.at[pl.ds((wb + w) * _WE, _WE)], cidx, jsem)

    def fire_data(w, ridx, gath, rad, isem, gs, rs):
        pltpu.make_async_copy(row_hbm.at[pl.ds(0, _WE)], ridx, isem).wait()
        pltpu.async_copy(h_hbm.at[ridx], gath, gs)
        pltpu.async_copy(rad_hbm.at[pl.ds((wb + w) * _WE, _WE), :], rad, rs)

    def wait_in(ridx, gath, rad, cidx, gs, rs, jsem):
        pltpu.make_async_copy(col_hbm.at[pl.ds(0, _WE)], cidx, jsem).wait()
        pltpu.make_async_copy(rad_hbm.at[pl.ds(0, _WE), :], rad, rs).wait()
        pltpu.make_async_copy(h_hbm.at[ridx], gath, gs).wait()

    ridx = (ridx0, ridx1, ridx2, ridx3)
    cidx = (cidx0, cidx1, cidx2, cidx3)
    isem = (is0, is1, is2, is3)
    jsem = (js0, js1, js2, js3)
    gath = (gath0, gath1)
    rad = (rad0, rad1)
    gsem = (gs0, gs1)
    rsem = (rs0, rs1)

    for q in range(4):
        fire_idx(q, ridx[q], cidx[q], isem[q], jsem[q])
    for b in range(2):
        fire_data(b, ridx[b], gath[b], rad[b], isem[b], gsem[b], rsem[b])

    @pl.loop(0, _QUADS)
    def _quad(p):
        w0 = 4 * p
        for o in range(4):
            b = o % 2
            q = o
            qn = (o + 2) % 4
            wait_in(ridx[q], gath[b], rad[b], cidx[q],
                    gsem[b], rsem[b], jsem[q])
            _ew_binop(gath[b], gath[b], rad[b], mul_bf, rows=_WE, static=True)
            pltpu.sync_copy(gath[b], aggs.at[cidx[q]], add=True)
            @pl.when(p < _QUADS - 1)
            def _(o=o, q=q, w0=w0):
                pltpu.async_copy(
                    row_hbm.at[pl.ds((wb + w0 + o + 4) * _WE, _WE)],
                    ridx[q], isem[q])
                pltpu.async_copy(
                    col_hbm.at[pl.ds((wb + w0 + o + 4) * _WE, _WE)],
                    cidx[q], jsem[q])

            if o < 2:
                fire_data(w0 + o + 2, ridx[qn], gath[b], rad[b],
                          isem[qn], gsem[b], rsem[b])
            else:
                @pl.when(p < _QUADS - 1)
                def _(o=o, b=b, qn=qn, w0=w0):
                    fire_data(w0 + o + 2, ridx[qn], gath[b], rad[b],
                              isem[qn], gsem[b], rsem[b])

    @pl.when(wid < _NLEFTE)
    def _left():
        wi = _NKWE * _NW + wid
        pltpu.sync_copy(row_hbm.at[pl.ds(wi * _WE, _WE)], ridx0)
        pltpu.sync_copy(col_hbm.at[pl.ds(wi * _WE, _WE)], cidx0)
        pltpu.async_copy(h_hbm.at[ridx0], gath0, gs0).wait()
        pltpu.sync_copy(rad_hbm.at[pl.ds(wi * _WE, _WE), :], rad0)
        _ew_binop(gath0, gath0, rad0, mul_bf, rows=_WE, static=True)
        pltpu.sync_copy(gath0, aggs.at[cidx0], add=True)

    plsc.subcore_barrier()
    for off, sz in _ZCHUNKS:
        pltpu.sync_copy(aggs.at[pl.ds(base + off, sz), :],
                        out_hbm.at[c, pl.ds(base + off, sz), :])

    @pl.when(s == _NS - 1)
    def _wtail():
        pltpu.sync_copy(aggs.at[pl.ds(_NS * _RPS, 16), :],
                        out_hbm.at[c, pl.ds(_NS * _RPS, 16), :])


def _sc_edge_pass(h, radial, row, col):
    mesh = plsc.VectorSubcoreMesh(core_axis_name="c", subcore_axis_name="s")
    f = pl.kernel(
        _edge_body,
        out_type=jax.ShapeDtypeStruct((_NC, N, D), jnp.float32),
        mesh=mesh,
        scratch_types=[
            pltpu.VMEM_SHARED((N, D), jnp.float32),
        ] + [pltpu.VMEM((_WE,), jnp.int32)] * 8
          + [pltpu.VMEM((_WE, D), jnp.float32)] * 2
          + [pltpu.VMEM((_WE, D), jnp.bfloat16)] * 2
          + [pltpu.SemaphoreType.DMA] * 12,
    )
    return f(h, radial, row, col)


def kernel(x, edge_index, pos, edge_attr, batch, W0, b0,
           Wc1, bc1, Wn1, bn1, g1, be1,
           Wc2, bc2, Wn2, bn2, g2, be2,
           Wc3, bc3, Wn3, bn3, g3, be3):
    row = edge_index[0].astype(jnp.int32)
    col = edge_index[1].astype(jnp.int32)
    pos128 = jnp.pad(pos, ((0, 0), (0, D - 3)))
    dif = _sc_pos_diff(pos128, row, col)
    h = _tc_h0(x, W0, b0)
    r1, r2, r3 = _tc_radial(dif, (Wc1, Wc2, Wc3), (bc1, bc2, bc3))
    for radial, wn, bn, g, be in ((r1, Wn1, bn1, g1, be1),
                                  (r2, Wn2, bn2, g2, be2),
                                  (r3, Wn3, bn3, g3, be3)):
        agg = _sc_edge_pass(h, radial, row, col)
        h = _tc_dense(h, agg, wn, bn, g, be)
    return _tc_graphsum(h, batch)


# speedup vs baseline: 1.2164x; 1.0320x over previous
import functools

import jax
import jax.numpy as jnp
from jax import lax
from jax.experimental import pallas as pl
from jax.experimental.pallas import tpu as pltpu
from jax.experimental.pallas import tpu_sc as plsc

N = 10000
E = 320000
D = 128
NUM_GRAPHS = 64
PPAD = 16

_NC, _NS = 2, 16
_NW = _NC * _NS
_W = 128
_NWIN = E // _W
_NKW = 2496 // _NW
_PAIRS = _NKW // 2
_NLEFT = _NWIN - _NKW * _NW
_WE = 64
_NWINE = E // _WE
_NKWE = (_NWINE // _NW) & ~3
_QUADS = _NKWE // 4
_NLEFTE = _NWINE - _NKWE * _NW
_RPS = 624
_CHUNKS = ((0, 128), (128, 128), (256, 128), (384, 128), (512, 112))
_ZR = 64
_ZCHUNKS = tuple((k * 64, 64) for k in range(9)) + ((576, 48),)


def _h0_body(x_ref, w_ref, b_ref, o_ref):
    t = jnp.dot(x_ref[...], w_ref[...], preferred_element_type=jnp.float32)
    t = t + b_ref[...]
    o_ref[...] = t * jax.nn.sigmoid(t)


def _tc_h0(x, w0, b0):
    blk = 1000
    return pl.pallas_call(
        _h0_body,
        grid=(N // blk,),
        in_specs=[
            pl.BlockSpec((blk, D), lambda i: (i, 0)),
            pl.BlockSpec((D, D), lambda i: (0, 0)),
            pl.BlockSpec((1, D), lambda i: (0, 0)),
        ],
        out_specs=pl.BlockSpec((blk, D), lambda i: (i, 0)),
        out_shape=jax.ShapeDtypeStruct((N, D), jnp.float32),
    )(x, w0, b0.reshape(1, D))


def _radial_body(n, *refs):
    dif_ref = refs[0]
    d = dif_ref[...]
    d2 = jnp.sum(d * d, axis=1, keepdims=True)
    dist = jnp.sqrt(d2 + 1e-12)
    mu = lax.broadcasted_iota(jnp.int32, (1, PPAD), 1).astype(jnp.float32) * 0.75
    sig = 6.0 / 9.0
    z = (dist - mu) / sig
    rbf = jnp.exp(-(z * z))
    for k in range(n):
        w_ref, b_ref, o_ref = refs[1 + 2 * k], refs[2 + 2 * k], refs[1 + 2 * n + k]
        t = jnp.dot(rbf, w_ref[...], preferred_element_type=jnp.float32)
        t = t + b_ref[...]
        o_ref[...] = (t * jax.nn.sigmoid(t)).astype(jnp.bfloat16)


def _tc_radial(dif, wc, bc):
    blk = 2000
    n = len(wc)
    wcp = [jnp.pad(w, ((0, PPAD - 9), (0, 0))) for w in wc]
    args = [dif]
    for w, b in zip(wcp, bc):
        args += [w, b.reshape(1, D)]
    outs = pl.pallas_call(
        functools.partial(_radial_body, n),
        grid=(E // blk,),
        in_specs=[
            pl.BlockSpec((blk, PPAD), lambda i: (i, 0)),
        ] + [pl.BlockSpec((PPAD, D), lambda i: (0, 0)),
             pl.BlockSpec((1, D), lambda i: (0, 0))] * n,
        out_specs=[pl.BlockSpec((blk, D), lambda i: (i, 0))] * n,
        out_shape=[jax.ShapeDtypeStruct((E, D), jnp.bfloat16)] * n,
    )(*args)
    return outs


def _dense_body(h_ref, a_ref, wn_ref, bn_ref, g_ref, be_ref, o_ref):
    t = h_ref[...] + a_ref[0] + a_ref[1]
    t = jnp.dot(t, wn_ref[...], preferred_element_type=jnp.float32)
    t = t + bn_ref[...]
    t = jnp.where(t >= 0, t, 0.01 * t)
    m = jnp.mean(t, axis=0, keepdims=True)
    c = t - m
    v = jnp.mean(c * c, axis=0, keepdims=True)
    o_ref[...] = c / jnp.sqrt(v + 1e-5) * g_ref[...] + be_ref[...]


def _tc_dense(h, agg, wn, bn, g, be):
    return pl.pallas_call(
        _dense_body,
        in_specs=[pl.BlockSpec((N, D), lambda: (0, 0)),
                  pl.BlockSpec((2, N, D), lambda: (0, 0, 0)),
                  pl.BlockSpec((D, D), lambda: (0, 0)),
                  pl.BlockSpec((1, D), lambda: (0, 0)),
                  pl.BlockSpec((1, D), lambda: (0, 0)),
                  pl.BlockSpec((1, D), lambda: (0, 0))],
        out_specs=pl.BlockSpec((N, D), lambda: (0, 0)),
        out_shape=jax.ShapeDtypeStruct((N, D), jnp.float32),
    )(h, agg, wn, bn.reshape(1, D), g.reshape(1, D), be.reshape(1, D))


def _graphsum_body(h_ref, b_ref, o_ref):
    onehot = (b_ref[...] == lax.broadcasted_iota(jnp.int32, (N, NUM_GRAPHS),
                                                 1)).astype(jnp.float32)
    o_ref[...] = lax.dot_general(onehot, h_ref[...], (((0,), (0,)), ((), ())),
                                 preferred_element_type=jnp.float32)


def _tc_graphsum(h, batch):
    return pl.pallas_call(
        _graphsum_body,
        in_specs=[pl.BlockSpec((N, D), lambda: (0, 0)),
                  pl.BlockSpec((N, 1), lambda: (0, 0))],
        out_specs=pl.BlockSpec((NUM_GRAPHS, D), lambda: (0, 0)),
        out_shape=jax.ShapeDtypeStruct((NUM_GRAPHS, D), jnp.float32),
    )(h, batch.reshape(N, 1))


def _ew_binop(dst, a, b, op, rows=_W, cols=D, static=False):
    if static:
        for r in range(rows):
            for j in range(cols // 16):
                sl = (r, pl.ds(j * 16, 16))
                dst[sl] = op(a[sl], b[sl])
        return

    @pl.loop(0, rows, unroll=4)
    def _rows(r):
        for j in range(cols // 16):
            sl = (r, pl.ds(j * 16, 16))
            dst[sl] = op(a[sl], b[sl])


def _posg_body(pos_hbm, row_hbm, col_hbm, od_hbm,
               ridx_all, cidx_all, bufr0, bufr1, bufc0, bufc1, dbuf0, dbuf1,
               rs0, rs1, cs0, cs1, ws0, ws1):
    c = lax.axis_index("c")
    s = lax.axis_index("s")
    wid = s * _NC + c
    wb = wid * _NKW
    eb = wb * _W

    pltpu.sync_copy(row_hbm.at[pl.ds(eb, _NKW * _W)], ridx_all)
    pltpu.sync_copy(col_hbm.at[pl.ds(eb, _NKW * _W)], cidx_all)

    def fire(w, br, bc, rs, cs):
        idx_r = ridx_all.at[pl.ds(w * _W, _W)]
        idx_c = cidx_all.at[pl.ds(w * _W, _W)]
        pltpu.async_copy(pos_hbm.at[idx_r], br, rs)
        pltpu.async_copy(pos_hbm.at[idx_c], bc, cs)

    def wait_g(br, bc, rs, cs):
        pltpu.make_async_copy(pos_hbm.at[ridx_all.at[pl.ds(0, _W)]], br,
                              rs).wait()
        pltpu.make_async_copy(pos_hbm.at[ridx_all.at[pl.ds(0, _W)]], bc,
                              cs).wait()

    def wait_w(db, ws):
        pltpu.make_async_copy(db, od_hbm.at[pl.ds(0, _W), :], ws).wait()

    fire(0, bufr0, bufc0, rs0, cs0)

    @pl.loop(0, _PAIRS)
    def _pair(p):
        w0 = 2 * p
        fire(w0 + 1, bufr1, bufc1, rs1, cs1)
        wait_g(bufr0, bufc0, rs0, cs0)

        @pl.when(p > 0)
        def _():
            wait_w(dbuf0, ws0)

        _ew_binop(dbuf0, bufr0, bufc0, lax.sub, cols=PPAD)
        pltpu.async_copy(dbuf0, od_hbm.at[pl.ds((wb + w0) * _W, _W), :], ws0)

        @pl.when(p < _PAIRS - 1)
        def _():
            fire(w0 + 2, bufr0, bufc0, rs0, cs0)

        wait_g(bufr1, bufc1, rs1, cs1)

        @pl.when(p > 0)
        def _():
            wait_w(dbuf1, ws1)

        _ew_binop(dbuf1, bufr1, bufc1, lax.sub, cols=PPAD)
        pltpu.async_copy(dbuf1, od_hbm.at[pl.ds((wb + w0 + 1) * _W, _W), :],
                         ws1)

    wait_w(dbuf0, ws0)
    wait_w(dbuf1, ws1)

    @pl.when(wid < _NLEFT)
    def _left():
        wi = _NKW * _NW + wid
        pltpu.sync_copy(row_hbm.at[pl.ds(wi * _W, _W)],
                        ridx_all.at[pl.ds(0, _W)])
        pltpu.sync_copy(col_hbm.at[pl.ds(wi * _W, _W)],
                        cidx_all.at[pl.ds(0, _W)])
        fire(0, bufr0, bufc0, rs0, cs0)
        wait_g(bufr0, bufc0, rs0, cs0)
        _ew_binop(dbuf0, bufr0, bufc0, lax.sub, cols=PPAD)
        pltpu.sync_copy(dbuf0, od_hbm.at[pl.ds(wi * _W, _W), :])


def _sc_pos_diff(pos128, row, col):
    mesh = plsc.VectorSubcoreMesh(core_axis_name="c", subcore_axis_name="s")
    f = pl.kernel(
        _posg_body,
        out_type=jax.ShapeDtypeStruct((E, PPAD), jnp.float32),
        mesh=mesh,
        scratch_types=[
            pltpu.VMEM((_NKW * _W,), jnp.int32),
            pltpu.VMEM((_NKW * _W,), jnp.int32),
        ] + [pltpu.VMEM((_W, D), jnp.float32)] * 4
          + [pltpu.VMEM((_W, PPAD), jnp.float32)] * 2
          + [pltpu.SemaphoreType.DMA] * 6,
    )
    return f(pos128, row, col)


def _edge_body(h_hbm, rad_hbm, row_hbm, col_hbm, out_hbm,
               aggs, ridx0, ridx1, ridx2, ridx3, cidx0, cidx1, cidx2, cidx3,
               gath0, gath1, rad0, rad1,
               gs0, gs1, rs0, rs1, is0, is1, is2, is3, js0, js1, js2, js3):
    c = lax.axis_index("c")
    s = lax.axis_index("s")
    wid = s * _NC + c
    wb = wid * _NKWE

    z16 = jnp.zeros((16,), jnp.float32)

    def mul_bf(x, y):
        return x * y.astype(jnp.float32)

    @pl.loop(0, _WE)
    def _zb(r):
        for j in range(8):
            gath0[r, pl.ds(j * 16, 16)] = z16

    base = s * _RPS
    for off, sz in _ZCHUNKS:
        pltpu.sync_copy(gath0.at[pl.ds(0, sz), :],
                        aggs.at[pl.ds(base + off, sz), :])

    @pl.when(s == _NS - 1)
    def _ztail():
        pltpu.sync_copy(gath0.at[pl.ds(0, 16), :],
                        aggs.at[pl.ds(_NS * _RPS, 16), :])

    plsc.subcore_barrier()

    def fire_idx(w, ridx, cidx, isem, jsem):
        pltpu.async_copy(row_hbm.at[pl.ds((wb + w) * _WE, _WE)], ridx, isem)
        pltpu.async_copy(col_hbm.at[pl.ds((wb + w) * _WE, _WE)], cidx, jsem)

    def fire_data(w, ridx, gath, rad, isem, gs, rs):
        pltpu.make_async_copy(row_hbm.at[pl.ds(0, _WE)], ridx, isem).wait()
        pltpu.async_copy(h_hbm.at[ridx], gath, gs)
        pltpu.async_copy(rad_hbm.at[pl.ds((wb + w) * _WE, _WE), :], rad, rs)

    def wait_in(ridx, gath, rad, cidx, gs, rs, jsem):
        pltpu.make_async_copy(col_hbm.at[pl.ds(0, _WE)], cidx, jsem).wait()
        pltpu.make_async_copy(rad_hbm.at[pl.ds(0, _WE), :], rad, rs).wait()
        pltpu.make_async_copy(h_hbm.at[ridx], gath, gs).wait()

    ridx = (ridx0, ridx1, ridx2, ridx3)
    cidx = (cidx0, cidx1, cidx2, cidx3)
    isem = (is0, is1, is2, is3)
    jsem = (js0, js1, js2, js3)
    gath = (gath0, gath1)
    rad = (rad0, rad1)
    gsem = (gs0, gs1)
    rsem = (rs0, rs1)

    for q in range(4):
        fire_idx(q, ridx[q], cidx[q], isem[q], jsem[q])
    for b in range(2):
        fire_data(b, ridx[b], gath[b], rad[b], isem[b], gsem[b], rsem[b])

    @pl.loop(0, _QUADS)
    def _quad(p):
        w0 = 4 * p
        for o in range(4):
            b = o % 2
            q = o
            qn = (o + 2) % 4
            wait_in(ridx[q], gath[b], rad[b], cidx[q],
                    gsem[b], rsem[b], jsem[q])
            _ew_binop(gath[b], gath[b], rad[b], mul_bf, rows=_WE, static=True)
            pltpu.sync_copy(gath[b], aggs.at[cidx[q]], add=True)
            @pl.when(p < _QUADS - 1)
            def _(o=o, q=q, w0=w0):
                pltpu.async_copy(
                    row_hbm.at[pl.ds((wb + w0 + o + 4) * _WE, _WE)],
                    ridx[q], isem[q])
                pltpu.async_copy(
                    col_hbm.at[pl.ds((wb + w0 + o + 4) * _WE, _WE)],
                    cidx[q], jsem[q])

            if o < 2:
                fire_data(w0 + o + 2, ridx[qn], gath[b], rad[b],
                          isem[qn], gsem[b], rsem[b])
            else:
                @pl.when(p < _QUADS - 1)
                def _(o=o, b=b, qn=qn, w0=w0):
                    fire_data(w0 + o + 2, ridx[qn], gath[b], rad[b],
                              isem[qn], gsem[b], rsem[b])

    @pl.when(wid < _NLEFTE)
    def _left():
        wi = _NKWE * _NW + wid
        pltpu.sync_copy(row_hbm.at[pl.ds(wi * _WE, _WE)], ridx0)
        pltpu.sync_copy(col_hbm.at[pl.ds(wi * _WE, _WE)], cidx0)
        pltpu.async_copy(h_hbm.at[ridx0], gath0, gs0).wait()
        pltpu.sync_copy(rad_hbm.at[pl.ds(wi * _WE, _WE), :], rad0)
        _ew_binop(gath0, gath0, rad0, mul_bf, rows=_WE, static=True)
        pltpu.sync_copy(gath0, aggs.at[cidx0], add=True)

    plsc.subcore_barrier()
    for off, sz in _ZCHUNKS:
        pltpu.sync_copy(aggs.at[pl.ds(base + off, sz), :],
                        out_hbm.at[c, pl.ds(base + off, sz), :])

    @pl.when(s == _NS - 1)
    def _wtail():
        pltpu.sync_copy(aggs.at[pl.ds(_NS * _RPS, 16), :],
                        out_hbm.at[c, pl.ds(_NS * _RPS, 16), :])


def _sc_edge_pass(h, radial, row, col):
    mesh = plsc.VectorSubcoreMesh(core_axis_name="c", subcore_axis_name="s")
    f = pl.kernel(
        _edge_body,
        out_type=jax.ShapeDtypeStruct((_NC, N, D), jnp.float32),
        mesh=mesh,
        scratch_types=[
            pltpu.VMEM_SHARED((N, D), jnp.float32),
        ] + [pltpu.VMEM((_WE,), jnp.int32)] * 8
          + [pltpu.VMEM((_WE, D), jnp.float32)] * 2
          + [pltpu.VMEM((_WE, D), jnp.bfloat16)] * 2
          + [pltpu.SemaphoreType.DMA] * 12,
    )
    return f(h, radial, row, col)


def kernel(x, edge_index, pos, edge_attr, batch, W0, b0,
           Wc1, bc1, Wn1, bn1, g1, be1,
           Wc2, bc2, Wn2, bn2, g2, be2,
           Wc3, bc3, Wn3, bn3, g3, be3):
    row = edge_index[0].astype(jnp.int32)
    col = edge_index[1].astype(jnp.int32)
    pos128 = jnp.pad(pos, ((0, 0), (0, D - 3)))
    dif = _sc_pos_diff(pos128, row, col)
    h = _tc_h0(x, W0, b0)
    (r1,) = _tc_radial(dif, (Wc1,), (bc1,))
    r2, r3 = _tc_radial(dif, (Wc2, Wc3), (bc2, bc3))
    for radial, wn, bn, g, be in ((r1, Wn1, bn1, g1, be1),
                                  (r2, Wn2, bn2, g2, be2),
                                  (r3, Wn3, bn3, g3, be3)):
        agg = _sc_edge_pass(h, radial, row, col)
        h = _tc_dense(h, agg, wn, bn, g, be)
    return _tc_graphsum(h, batch)
